# baseline (speedup 1.0000x reference)
import functools

import jax
import jax.numpy as jnp
from jax import lax
from jax.experimental import pallas as pl
from jax.experimental.pallas import tpu as pltpu
from jax.experimental.pallas import tpu_sc as plsc

_NC = 2
_NS = 16
_NW = _NC * _NS
_LANES = 16

_CH = 400
_GATHER = 80


def _adapter_body(img_ref, w_ref, b_ref, out_ref):
    x = img_ref[0]
    w = w_ref[...]
    acc = lax.dot_general(
        x, w,
        dimension_numbers=(((0,), (1,)), ((), ())),
        preferred_element_type=jnp.float32,
    )
    out_ref[0] = acc + b_ref[...]


def _adapter_pair_body(img_ref, w_ref, b_ref, out_ref):
    x = img_ref[0]
    w = w_ref[...]
    t_half = x.shape[1] // 2
    y = lax.dot_general(
        x, w,
        dimension_numbers=(((0,), (1,)), ((), ())),
        preferred_element_type=jnp.float32,
    ) + b_ref[...]
    out_ref[0] = jnp.concatenate([y[:t_half], y[t_half:]], axis=1)


def _adapter_pair(img, w, b, tile):
    B, C, P = img.shape
    grid = (B, P // tile)
    return pl.pallas_call(
        _adapter_pair_body,
        grid=grid,
        in_specs=[
            pl.BlockSpec((1, C, tile), lambda bb, t: (bb, 0, t)),
            pl.BlockSpec((C, C), lambda bb, t: (0, 0)),
            pl.BlockSpec((1, C), lambda bb, t: (0, 0)),
        ],
        out_specs=pl.BlockSpec((1, tile // 2, 2 * C), lambda bb, t: (bb, t, 0)),
        out_shape=jax.ShapeDtypeStruct((B, P // 2, 2 * C), jnp.float32),
        compiler_params=pltpu.CompilerParams(fuse_transposed_lhs_in_matmul=True),
    )(img, w, b.reshape(1, C))


def _adapter(img, w, b, tile):
    B, C, P = img.shape
    O = w.shape[0]
    grid = (B, P // tile)
    return pl.pallas_call(
        _adapter_body,
        grid=grid,
        in_specs=[
            pl.BlockSpec((1, C, tile), lambda bb, t: (bb, 0, t)),
            pl.BlockSpec((O, C), lambda bb, t: (0, 0)),
            pl.BlockSpec((1, O), lambda bb, t: (0, 0)),
        ],
        out_specs=pl.BlockSpec((1, tile, O), lambda bb, t: (bb, t, 0)),
        out_shape=jax.ShapeDtypeStruct((B, P, O), jnp.float32),
    )(img, w, b.reshape(1, O))


def _idx_body(vc_ref, out_ref, *, W, H, table_rows_per_batch, pair_tile, tb):
    half_shift = pair_tile.bit_length() - 2
    half_mask = (pair_tile // 2) - 1
    block_mask = ~(pair_tile - 1)
    b = pl.program_id(0)
    t = pl.program_id(1)
    c = jnp.clip(vc_ref[0], 0, W - 1)
    ii = lax.broadcasted_iota(jnp.int32, (1, 3), 1)
    wrow = jnp.where(ii == 0, 1, jnp.where(ii == 1, W, 0)).astype(jnp.int32)
    p = jnp.sum(c * wrow, axis=1)
    row = ((p & block_mask)
           | ((p & half_mask) << 1)
           | ((p >> half_shift) & 1))
    out_ref[b, pl.ds(t * tb, tb)] = row + b * table_rows_per_batch


def _voxel_idx(voxel_coords, W, H, table_rows_per_batch, pair_tile, tb):
    B, N, _ = voxel_coords.shape
    body = functools.partial(
        _idx_body, W=W, H=H,
        table_rows_per_batch=table_rows_per_batch, pair_tile=pair_tile, tb=tb)
    return pl.pallas_call(
        body,
        grid=(B, N // tb),
        in_specs=[pl.BlockSpec((1, tb, 3), lambda bb, t: (bb, t, 0))],
        out_specs=pl.BlockSpec((B, N), lambda bb, t: (0, 0)),
        out_shape=jax.ShapeDtypeStruct((B, N), jnp.int32),
    )(voxel_coords)


def _sc_gather(table, idx):
    B, N = idx.shape
    C = table.shape[1]
    assert N % _CH == 0
    cpb = N // _CH
    nchunk = B * cpb
    iters = -(-nchunk // _NW)
    mesh = plsc.VectorSubcoreMesh(core_axis_name="c", subcore_axis_name="s")

    @functools.partial(
        pl.kernel,
        mesh=mesh,
        out_type=jax.ShapeDtypeStruct((B * N, C), jnp.float32),
        scratch_types=[
            pltpu.VMEM((_CH,), jnp.int32),
            pltpu.VMEM((_CH, C), jnp.float32),
            pltpu.SemaphoreType.DMA,
        ],
        compiler_params=pltpu.CompilerParams(
            use_tc_tiling_on_sc=False, needs_layout_passes=False),
    )
    def k(idx_hbm, table_hbm, out_hbm, idxv, rows, sem):
        wid = lax.axis_index("s") * _NC + lax.axis_index("c")

        def chunk(i, carry):
            c = wid + i * _NW

            @pl.when(c < nchunk)
            def _():
                b = c // cpb
                n0 = (c - b * cpb) * _CH
                pltpu.sync_copy(idx_hbm.at[b, pl.ds(n0, _CH)], idxv)
                copies = [
                    pltpu.async_copy(
                        table_hbm.at[idxv.at[pl.ds(kk * _GATHER, _GATHER)]],
                        rows.at[pl.ds(kk * _GATHER, _GATHER)],
                        sem,
                    )
                    for kk in range(_CH // _GATHER)
                ]
                for cp in copies:
                    cp.wait()
                pltpu.sync_copy(rows, out_hbm.at[pl.ds(b * N + n0, _CH)])

            return carry

        lax.fori_loop(0, iters, chunk, 0)

    return k(idx, table)


def kernel(img_features, projection_matrix, voxel_coords, W_adapter, b_adapter):
    del projection_matrix
    B, C, H, W = img_features.shape
    N = voxel_coords.shape[1]
    P = H * W

    img = img_features.reshape(B, C, P)
    tile = 16384
    feats = _adapter_pair(img, W_adapter, b_adapter, tile=tile)
    table = feats.reshape(B * P, C)

    vc = voxel_coords.astype(jnp.int32)
    half_shift = tile.bit_length() - 2
    half_mask = (tile // 2) - 1
    block_mask = ~(tile - 1)
    w3 = jnp.array([1, W, 0], jnp.int32)
    p = jnp.einsum('bnk,k->bn', jnp.clip(vc, 0, W - 1), w3)
    row = ((p & block_mask) | ((p & half_mask) << 1) | ((p >> half_shift) & 1))
    idx = row + jnp.arange(B, dtype=jnp.int32)[:, None] * P

    out = _sc_gather(table, idx)
    return out.reshape(B, N, C)


# speedup vs baseline: 1.1068x; 1.1068x over previous
import functools

import jax
import jax.numpy as jnp
from jax import lax
from jax.experimental import pallas as pl
from jax.experimental.pallas import tpu as pltpu
from jax.experimental.pallas import tpu_sc as plsc

_NC = 2
_NS = 16
_NW = _NC * _NS
_LANES = 16

_CH = 640
_GATHER = 128


def _adapter_body(img_ref, w_ref, b_ref, out_ref):
    x = img_ref[0]
    w = w_ref[...]
    acc = lax.dot_general(
        x, w,
        dimension_numbers=(((0,), (1,)), ((), ())),
        preferred_element_type=jnp.float32,
    )
    out_ref[0] = acc + b_ref[...]


def _adapter_pair_body(img_ref, w_ref, b_ref, out_ref):
    x = img_ref[0]
    w = w_ref[...]
    t_half = x.shape[1] // 2
    y = lax.dot_general(
        x, w,
        dimension_numbers=(((0,), (1,)), ((), ())),
        preferred_element_type=jnp.float32,
    ) + b_ref[...]
    out_ref[0] = jnp.concatenate([y[:t_half], y[t_half:]], axis=1)


def _adapter_pair(img, w, b, tile):
    B, C, P = img.shape
    grid = (B, P // tile)
    return pl.pallas_call(
        _adapter_pair_body,
        grid=grid,
        in_specs=[
            pl.BlockSpec((1, C, tile), lambda bb, t: (bb, 0, t)),
            pl.BlockSpec((C, C), lambda bb, t: (0, 0)),
            pl.BlockSpec((1, C), lambda bb, t: (0, 0)),
        ],
        out_specs=pl.BlockSpec((1, tile // 2, 2 * C), lambda bb, t: (bb, t, 0)),
        out_shape=jax.ShapeDtypeStruct((B, P // 2, 2 * C), jnp.float32),
    )(img, w, b.reshape(1, C))


def _adapter(img, w, b, tile):
    B, C, P = img.shape
    O = w.shape[0]
    grid = (B, P // tile)
    return pl.pallas_call(
        _adapter_body,
        grid=grid,
        in_specs=[
            pl.BlockSpec((1, C, tile), lambda bb, t: (bb, 0, t)),
            pl.BlockSpec((O, C), lambda bb, t: (0, 0)),
            pl.BlockSpec((1, O), lambda bb, t: (0, 0)),
        ],
        out_specs=pl.BlockSpec((1, tile, O), lambda bb, t: (bb, t, 0)),
        out_shape=jax.ShapeDtypeStruct((B, P, O), jnp.float32),
    )(img, w, b.reshape(1, O))


def _idx_body(vc_ref, out_ref, *, W, H, table_rows_per_batch, pair_tile, tb):
    half_shift = pair_tile.bit_length() - 2
    half_mask = (pair_tile // 2) - 1
    block_mask = ~(pair_tile - 1)
    b = pl.program_id(0)
    t = pl.program_id(1)
    c = jnp.clip(vc_ref[0], 0, W - 1)
    ii = lax.broadcasted_iota(jnp.int32, (1, 3), 1)
    wrow = jnp.where(ii == 0, 1, jnp.where(ii == 1, W, 0)).astype(jnp.int32)
    p = jnp.sum(c * wrow, axis=1)
    row = ((p & block_mask)
           | ((p & half_mask) << 1)
           | ((p >> half_shift) & 1))
    out_ref[b, pl.ds(t * tb, tb)] = row + b * table_rows_per_batch


def _voxel_idx(voxel_coords, W, H, table_rows_per_batch, pair_tile, tb):
    B, N, _ = voxel_coords.shape
    body = functools.partial(
        _idx_body, W=W, H=H,
        table_rows_per_batch=table_rows_per_batch, pair_tile=pair_tile, tb=tb)
    return pl.pallas_call(
        body,
        grid=(B, N // tb),
        in_specs=[pl.BlockSpec((1, tb, 3), lambda bb, t: (bb, t, 0))],
        out_specs=pl.BlockSpec((B, N), lambda bb, t: (0, 0)),
        out_shape=jax.ShapeDtypeStruct((B, N), jnp.int32),
    )(voxel_coords)


def _sc_gather(table, idx):
    total = idx.shape[0]
    C = table.shape[1]
    assert total % _CH == 0
    nchunk = total // _CH
    iters = -(-nchunk // _NW)
    mesh = plsc.VectorSubcoreMesh(core_axis_name="c", subcore_axis_name="s")

    @functools.partial(
        pl.kernel,
        mesh=mesh,
        out_type=jax.ShapeDtypeStruct((total, C), jnp.float32),
        scratch_types=[
            pltpu.VMEM((_CH,), jnp.int32),
            pltpu.VMEM((_CH,), jnp.int32),
            pltpu.VMEM((_CH, C), jnp.float32),
            pltpu.VMEM((_CH, C), jnp.float32),
            pltpu.SemaphoreType.DMA,
            pltpu.SemaphoreType.DMA,
        ],
        compiler_params=pltpu.CompilerParams(
            use_tc_tiling_on_sc=False, needs_layout_passes=False),
    )
    def k(idx_hbm, table_hbm, out_hbm, idxv0, idxv1, rows0, rows1,
          sem_g, sem_o):
        wid = lax.axis_index("s") * _NC + lax.axis_index("c")

        def wait_one_out():
            pltpu.make_async_copy(
                rows0, out_hbm.at[pl.ds(0, _CH)], sem_o).wait()

        def do_chunk(i, c, idxv, rows):
            base = c * _CH
            pltpu.sync_copy(idx_hbm.at[pl.ds(base, _CH)], idxv)
            copies = [
                pltpu.async_copy(
                    table_hbm.at[idxv.at[pl.ds(kk * _GATHER, _GATHER)]],
                    rows.at[pl.ds(kk * _GATHER, _GATHER)],
                    sem_g,
                )
                for kk in range(_CH // _GATHER)
            ]

            @pl.when(i > 0)
            def _():
                wait_one_out()

            for cp in copies:
                cp.wait()
            pltpu.async_copy(rows, out_hbm.at[pl.ds(base, _CH)], sem_o)

        def chunk(i, carry):
            c = wid + i * _NW

            @pl.when(c < nchunk)
            def _():
                @pl.when(i % 2 == 0)
                def _():
                    do_chunk(i, c, idxv0, rows0)

                @pl.when(i % 2 == 1)
                def _():
                    do_chunk(i, c, idxv1, rows1)

            return carry

        lax.fori_loop(0, iters, chunk, 0)

        @pl.when(wid < nchunk)
        def _():
            wait_one_out()

    return k(idx, table)


def kernel(img_features, projection_matrix, voxel_coords, W_adapter, b_adapter):
    del projection_matrix
    B, C, H, W = img_features.shape
    N = voxel_coords.shape[1]
    P = H * W

    img = img_features.reshape(B, C, P)
    tile = 16384
    feats = _adapter_pair(img, W_adapter, b_adapter, tile=tile)
    table = feats.reshape(B * P, C)

    vc = voxel_coords.astype(jnp.int32)
    half_shift = tile.bit_length() - 2
    half_mask = (tile // 2) - 1
    block_mask = ~(tile - 1)
    p = (jnp.clip(vc[:, :, 1], 0, H - 1) * W
         + jnp.clip(vc[:, :, 0], 0, W - 1))
    row = ((p & block_mask) | ((p & half_mask) << 1) | ((p >> half_shift) & 1))
    idx = (row + jnp.arange(B, dtype=jnp.int32)[:, None] * P).reshape(B * N)

    out = _sc_gather(table, idx)
    return out.reshape(B, N, C)
